# Initial kernel scaffold; baseline (speedup 1.0000x reference)
import functools

import jax
import jax.numpy as jnp
from jax import lax
from jax.experimental import pallas as pl
from jax.experimental.pallas import tpu as pltpu
from jax.experimental.pallas import tpu_sc as plsc

N = 10000
E = 320000
D_IN = 128
D_HID = 128
D_LAT = 32
BN_EPS = 1e-5

NC = 2
NS = 16
LANES = 16

EPW = E // (NC * NS)
CHUNK = 80
NCHUNK = EPW // CHUNK
ROWS_PER_SUB = 640
FULL_ZCHUNKS = ROWS_PER_SUB // CHUNK
LAST_ZCHUNKS = (N - (NS - 1) * ROWS_PER_SUB) // CHUNK


def _mesh():
  return plsc.VectorSubcoreMesh(
      core_axis_name="c", subcore_axis_name="s", num_cores=NC, num_subcores=NS)


def _deg_kernel(col, ew1):
  @functools.partial(
      pl.kernel,
      out_type=jax.ShapeDtypeStruct((NC, N, 1), jnp.float32),
      mesh=_mesh(),
      scratch_types=[
          pltpu.VMEM((CHUNK,), jnp.int32),
          pltpu.VMEM((CHUNK, 1), jnp.float32),
          pltpu.VMEM((CHUNK, 1), jnp.float32),
          pltpu.VMEM_SHARED((N, 1), jnp.float32),
      ],
  )
  def deg(col_hbm, ew_hbm, out_hbm, col_v, ew_v, zero_v, acc_sh):
    c = lax.axis_index("c")
    s = lax.axis_index("s")
    wid = s * NC + c

    def zfill(i, _):
      zero_v[pl.ds(i * LANES, LANES), 0] = jnp.zeros((LANES,), jnp.float32)
      return 0
    lax.fori_loop(0, CHUNK // LANES, zfill, 0)

    base_r = s * ROWS_PER_SUB
    nz = jnp.where(s == NS - 1, LAST_ZCHUNKS, FULL_ZCHUNKS)

    def zchunk(j, _):
      pltpu.sync_copy(zero_v, acc_sh.at[pl.ds(base_r + j * CHUNK, CHUNK)])
      return 0
    lax.fori_loop(0, nz, zchunk, 0)
    plsc.subcore_barrier()

    ebase = wid * EPW

    def body(i, _):
      b0 = ebase + i * CHUNK
      pltpu.sync_copy(col_hbm.at[pl.ds(b0, CHUNK)], col_v)
      pltpu.sync_copy(ew_hbm.at[pl.ds(b0, CHUNK)], ew_v)
      pltpu.sync_copy(ew_v, acc_sh.at[col_v], add=True)
      return 0
    lax.fori_loop(0, NCHUNK, body, 0)
    plsc.subcore_barrier()

    def ochunk(j, _):
      r0 = base_r + j * CHUNK
      pltpu.sync_copy(acc_sh.at[pl.ds(r0, CHUNK)], out_hbm.at[c, pl.ds(r0, CHUNK)])
      return 0
    lax.fori_loop(0, nz, ochunk, 0)

  return deg(col, ew1)


def _propagate(g, row, col, ew, d):
  @functools.partial(
      pl.kernel,
      out_type=jax.ShapeDtypeStruct((NC, N, d), jnp.float32),
      mesh=_mesh(),
      scratch_types=[
          pltpu.VMEM((CHUNK,), jnp.int32),
          pltpu.VMEM((CHUNK,), jnp.int32),
          pltpu.VMEM((CHUNK,), jnp.float32),
          pltpu.VMEM((CHUNK, d), jnp.float32),
          pltpu.VMEM_SHARED((N, d), jnp.float32),
          pltpu.SemaphoreType.DMA,
      ],
  )
  def prop(g_hbm, row_hbm, col_hbm, ew_hbm, out_hbm,
           row_v, col_v, ew_v, rows_v, acc_sh, sem):
    c = lax.axis_index("c")
    s = lax.axis_index("s")
    wid = s * NC + c

    def zrow(e, _):
      for j in range(d // LANES):
        rows_v[e, pl.ds(j * LANES, LANES)] = jnp.zeros((LANES,), jnp.float32)
      return 0
    lax.fori_loop(0, CHUNK, zrow, 0)

    base_r = s * ROWS_PER_SUB
    nz = jnp.where(s == NS - 1, LAST_ZCHUNKS, FULL_ZCHUNKS)

    def zchunk(j, _):
      pltpu.sync_copy(rows_v, acc_sh.at[pl.ds(base_r + j * CHUNK, CHUNK)])
      return 0
    lax.fori_loop(0, nz, zchunk, 0)
    plsc.subcore_barrier()

    ebase = wid * EPW

    def body(i, _):
      b0 = ebase + i * CHUNK
      pltpu.sync_copy(row_hbm.at[pl.ds(b0, CHUNK)], row_v)
      pltpu.sync_copy(col_hbm.at[pl.ds(b0, CHUNK)], col_v)
      pltpu.sync_copy(ew_hbm.at[pl.ds(b0, CHUNK)], ew_v)
      pltpu.async_copy(g_hbm.at[row_v], rows_v, sem).wait()

      def scale(e, _):
        w = plsc.load_gather(ew_v, [jnp.full((LANES,), e, jnp.int32)])
        for j in range(d // LANES):
          rows_v[e, pl.ds(j * LANES, LANES)] = (
              rows_v[e, pl.ds(j * LANES, LANES)] * w)
        return 0
      lax.fori_loop(0, CHUNK, scale, 0)

      pltpu.sync_copy(rows_v, acc_sh.at[col_v], add=True)
      return 0
    lax.fori_loop(0, NCHUNK, body, 0)
    plsc.subcore_barrier()

    def ochunk(j, _):
      r0 = base_r + j * CHUNK
      pltpu.sync_copy(acc_sh.at[pl.ds(r0, CHUNK)],
                      out_hbm.at[c, pl.ds(r0, CHUNK)])
      return 0
    lax.fori_loop(0, nz, ochunk, 0)

  return prop(g, row, col, ew)


_INV_SQRT_1EPS = float(1.0 / jnp.sqrt(jnp.float32(1.0 + BN_EPS)))


def _tc1(x, W0, dp):
  def body(x_ref, w_ref, dp_ref, hw_ref, g_ref, dinv_ref):
    hw = jnp.dot(x_ref[...], w_ref[...], preferred_element_type=jnp.float32)
    deg = dp_ref[0] + dp_ref[1] + 1.0
    dinv = jnp.where(deg > 0, lax.rsqrt(deg), 0.0)
    hw_ref[...] = hw
    g_ref[...] = hw * dinv
    dinv_ref[...] = dinv

  return pl.pallas_call(
      body,
      out_shape=[
          jax.ShapeDtypeStruct((N, D_HID), jnp.float32),
          jax.ShapeDtypeStruct((N, D_HID), jnp.float32),
          jax.ShapeDtypeStruct((N, 1), jnp.float32),
      ],
  )(x, W0, dp)


def _tc_mid(sp, hw, dinv, b, gamma, beta, Wnext, res=None):
  d_next = Wnext.shape[1]

  def body(sp_ref, hw_ref, dinv_ref, b_ref, ga_ref, be_ref, w_ref, *rest):
    if res is not None:
      res_ref, h_ref, hwn_ref, gn_ref = rest
    else:
      h_ref, hwn_ref, gn_ref = rest
    dinv = dinv_ref[...]
    conv = ((sp_ref[0] + sp_ref[1]) * dinv
            + hw_ref[...] * (dinv * dinv) + b_ref[...])
    h = conv * (ga_ref[...] * _INV_SQRT_1EPS) + be_ref[...]
    h = jnp.maximum(h, 0.0)
    if res is not None:
      h = h + res_ref[...]
    hwn = jnp.dot(h, w_ref[...], preferred_element_type=jnp.float32)
    h_ref[...] = h
    hwn_ref[...] = hwn
    gn_ref[...] = hwn * dinv

  args = [sp, hw, dinv, b, gamma, beta, Wnext]
  if res is not None:
    args.append(res)
  return pl.pallas_call(
      body,
      out_shape=[
          jax.ShapeDtypeStruct((N, D_HID), jnp.float32),
          jax.ShapeDtypeStruct((N, d_next), jnp.float32),
          jax.ShapeDtypeStruct((N, d_next), jnp.float32),
      ],
  )(*args)


def _tc_final(sp, hw2, dinv, bmv, eps):
  def body(sp_ref, hw_ref, dinv_ref, b_ref, eps_ref, qz_ref, qm_ref, qs_ref):
    dinv = dinv_ref[...]
    q = ((sp_ref[0] + sp_ref[1]) * dinv
         + hw_ref[...] * (dinv * dinv) + b_ref[...])
    qm = q[:, :D_LAT]
    qlv = q[:, D_LAT:]
    qs = jnp.exp(0.5 * qlv)
    qm_ref[...] = qm
    qs_ref[...] = qs
    qz_ref[...] = qm + qs * eps_ref[...]

  return pl.pallas_call(
      body,
      out_shape=[
          jax.ShapeDtypeStruct((N, D_LAT), jnp.float32),
          jax.ShapeDtypeStruct((N, D_LAT), jnp.float32),
          jax.ShapeDtypeStruct((N, D_LAT), jnp.float32),
      ],
  )(sp, hw2, dinv, bmv, eps)


def kernel(x, edge_index, edge_weight, W0, b0, gamma0, beta0,
           W1, b1, gamma1, beta1, Wm, bm, Wv, bv, eps):
  row = edge_index[0].astype(jnp.int32)
  col = edge_index[1].astype(jnp.int32)
  ew = edge_weight.astype(jnp.float32)
  ew1 = ew.reshape(E, 1)

  dp = _deg_kernel(col, ew1)

  hw0, g0, dinv = _tc1(x, W0, dp)
  s0 = _propagate(g0, row, col, ew, D_HID)

  h1, hw1, g1 = _tc_mid(s0, hw0, dinv, b0, gamma0, beta0, W1)
  s1 = _propagate(g1, row, col, ew, D_HID)

  Wmv = jnp.concatenate([Wm, Wv], axis=1)
  bmv = jnp.concatenate([bm, bv], axis=0)
  _, hw2, g2 = _tc_mid(s1, hw1, dinv, b1, gamma1, beta1, Wmv, res=h1)
  s2 = _propagate(g2, row, col, ew, 2 * D_LAT)

  q_z, q_m, q_s = _tc_final(s2, hw2, dinv, bmv, eps)
  return (q_z, q_m, q_s)


# speedup vs baseline: 9.5073x; 9.5073x over previous
import functools
import math

import jax
import jax.numpy as jnp
from jax import lax
from jax.experimental import pallas as pl
from jax.experimental.pallas import tpu as pltpu
from jax.experimental.pallas import tpu_sc as plsc

N = 10000
E = 320000
D_IN = 128
D_HID = 128
D_LAT = 32
BN_EPS = 1e-5

NC = 2
NS = 16
LANES = 16

EPW = E // (NC * NS)
CHUNK = 80
NCHUNK = EPW // CHUNK
ROWS_PER_SUB = 640
FULL_ZCHUNKS = ROWS_PER_SUB // CHUNK
LAST_ZCHUNKS = (N - (NS - 1) * ROWS_PER_SUB) // CHUNK


def _mesh():
  return plsc.VectorSubcoreMesh(
      core_axis_name="c", subcore_axis_name="s", num_cores=NC, num_subcores=NS)


def _deg_kernel(col, ew):
  @functools.partial(
      pl.kernel,
      out_type=jax.ShapeDtypeStruct((NC * N,), jnp.float32),
      mesh=_mesh(),
      scratch_types=[
          pltpu.VMEM((CHUNK,), jnp.int32),
          pltpu.VMEM((CHUNK,), jnp.float32),
          pltpu.VMEM((CHUNK,), jnp.float32),
          pltpu.VMEM_SHARED((N,), jnp.float32),
      ],
  )
  def deg(col_hbm, ew_hbm, out_hbm, col_v, ew_v, zero_v, acc_sh):
    c = lax.axis_index("c")
    s = lax.axis_index("s")
    wid = s * NC + c

    def zfill(i, _):
      zero_v[pl.ds(i * LANES, LANES)] = jnp.zeros((LANES,), jnp.float32)
      return 0
    lax.fori_loop(0, CHUNK // LANES, zfill, 0)

    base_r = s * ROWS_PER_SUB
    nz = jnp.where(s == NS - 1, LAST_ZCHUNKS, FULL_ZCHUNKS)

    def zchunk(j, _):
      pltpu.sync_copy(zero_v, acc_sh.at[pl.ds(base_r + j * CHUNK, CHUNK)])
      return 0
    lax.fori_loop(0, nz, zchunk, 0)
    plsc.subcore_barrier()

    ebase = wid * EPW

    def body(i, _):
      b0 = ebase + i * CHUNK
      pltpu.sync_copy(col_hbm.at[pl.ds(b0, CHUNK)], col_v)
      pltpu.sync_copy(ew_hbm.at[pl.ds(b0, CHUNK)], ew_v)
      pltpu.sync_copy(ew_v, acc_sh.at[col_v], add=True)
      return 0
    lax.fori_loop(0, NCHUNK, body, 0)
    plsc.subcore_barrier()

    def ochunk(j, _):
      r0 = base_r + j * CHUNK
      pltpu.sync_copy(acc_sh.at[pl.ds(r0, CHUNK)], ew_v)
      pltpu.sync_copy(ew_v, out_hbm.at[pl.ds(c * N + r0, CHUNK)])
      return 0
    lax.fori_loop(0, nz, ochunk, 0)

  return deg(col, ew)


def _propagate(g, row, col, ew, d):
  @functools.partial(
      pl.kernel,
      out_type=jax.ShapeDtypeStruct((NC * N, d), jnp.float32),
      mesh=_mesh(),
      scratch_types=[
          pltpu.VMEM((CHUNK,), jnp.int32),
          pltpu.VMEM((CHUNK,), jnp.int32),
          pltpu.VMEM((CHUNK,), jnp.float32),
          pltpu.VMEM((CHUNK, d), jnp.float32),
          pltpu.VMEM_SHARED((N, d), jnp.float32),
          pltpu.SemaphoreType.DMA,
      ],
  )
  def prop(g_hbm, row_hbm, col_hbm, ew_hbm, out_hbm,
           row_v, col_v, ew_v, rows_v, acc_sh, sem):
    c = lax.axis_index("c")
    s = lax.axis_index("s")
    wid = s * NC + c

    def zrow(e, _):
      for j in range(d // LANES):
        rows_v[e, pl.ds(j * LANES, LANES)] = jnp.zeros((LANES,), jnp.float32)
      return 0
    lax.fori_loop(0, CHUNK, zrow, 0)

    base_r = s * ROWS_PER_SUB
    nz = jnp.where(s == NS - 1, LAST_ZCHUNKS, FULL_ZCHUNKS)

    def zchunk(j, _):
      pltpu.sync_copy(rows_v, acc_sh.at[pl.ds(base_r + j * CHUNK, CHUNK)])
      return 0
    lax.fori_loop(0, nz, zchunk, 0)
    plsc.subcore_barrier()

    ebase = wid * EPW

    def body(i, _):
      b0 = ebase + i * CHUNK
      pltpu.sync_copy(row_hbm.at[pl.ds(b0, CHUNK)], row_v)
      pltpu.sync_copy(col_hbm.at[pl.ds(b0, CHUNK)], col_v)
      pltpu.sync_copy(ew_hbm.at[pl.ds(b0, CHUNK)], ew_v)
      pltpu.async_copy(g_hbm.at[row_v], rows_v, sem).wait()

      def scale(grp, _):
        wv = ew_v[pl.ds(grp * LANES, LANES)]
        for l in range(LANES):
          w = wv[l]
          e = grp * LANES + l
          for j in range(d // LANES):
            rows_v[e, pl.ds(j * LANES, LANES)] = (
                rows_v[e, pl.ds(j * LANES, LANES)] * w)
        return 0
      lax.fori_loop(0, CHUNK // LANES, scale, 0)

      pltpu.sync_copy(rows_v, acc_sh.at[col_v], add=True)
      return 0
    lax.fori_loop(0, NCHUNK, body, 0)
    plsc.subcore_barrier()

    def ochunk(j, _):
      r0 = base_r + j * CHUNK
      pltpu.sync_copy(acc_sh.at[pl.ds(r0, CHUNK)], rows_v)
      pltpu.sync_copy(rows_v, out_hbm.at[pl.ds(c * N + r0, CHUNK)])
      return 0
    lax.fori_loop(0, nz, ochunk, 0)

  return prop(g, row, col, ew).reshape(NC, N, d)


_INV_SQRT_1EPS = 1.0 / math.sqrt(1.0 + BN_EPS)


def _tc1(x, W0, dp):
  def body(x_ref, w_ref, dp_ref, hw_ref, g_ref, dinv_ref):
    hw = jnp.dot(x_ref[...], w_ref[...], preferred_element_type=jnp.float32)
    deg = dp_ref[0] + dp_ref[1] + 1.0
    dinv = jnp.where(deg > 0, lax.rsqrt(deg), 0.0)
    hw_ref[...] = hw
    g_ref[...] = hw * dinv
    dinv_ref[...] = dinv

  return pl.pallas_call(
      body,
      out_shape=[
          jax.ShapeDtypeStruct((N, D_HID), jnp.float32),
          jax.ShapeDtypeStruct((N, D_HID), jnp.float32),
          jax.ShapeDtypeStruct((N, 1), jnp.float32),
      ],
  )(x, W0, dp)


def _tc_mid(sp, hw, dinv, b, gamma, beta, Wnext=None, res=None):
  d_next = D_HID if Wnext is None else Wnext.shape[1]

  def body(*refs):
    it = iter(refs)
    sp_ref, hw_ref, dinv_ref, b_ref, ga_ref, be_ref = (next(it) for _ in range(6))
    w_ref = next(it) if Wnext is not None else None
    res_ref = next(it) if res is not None else None
    h_ref, hwn_ref, gn_ref = next(it), next(it), next(it)
    dinv = dinv_ref[...]
    conv = ((sp_ref[0] + sp_ref[1]) * dinv
            + hw_ref[...] * (dinv * dinv) + b_ref[...])
    h = conv * (ga_ref[...] * _INV_SQRT_1EPS) + be_ref[...]
    h = jnp.maximum(h, 0.0)
    if res is not None:
      h = h + res_ref[...]
    if Wnext is not None:
      hwn = jnp.dot(h, w_ref[...], preferred_element_type=jnp.float32)
    else:
      hwn = h
    h_ref[...] = h
    hwn_ref[...] = hwn
    gn_ref[...] = hwn * dinv

  args = [sp, hw, dinv, b, gamma, beta]
  if Wnext is not None:
    args.append(Wnext)
  if res is not None:
    args.append(res)
  return pl.pallas_call(
      body,
      out_shape=[
          jax.ShapeDtypeStruct((N, D_HID), jnp.float32),
          jax.ShapeDtypeStruct((N, d_next), jnp.float32),
          jax.ShapeDtypeStruct((N, d_next), jnp.float32),
      ],
  )(*args)


def _tc_final(sp, h2, dinv, Wmv, bmv, eps):
  def body(sp_ref, h_ref, dinv_ref, w_ref, b_ref, eps_ref,
           qz_ref, qm_ref, qs_ref):
    dinv = dinv_ref[...]
    sfull = (sp_ref[0] + sp_ref[1]) * dinv + h_ref[...] * (dinv * dinv)
    q = jnp.dot(sfull, w_ref[...], preferred_element_type=jnp.float32) + b_ref[...]
    qm = q[:, :D_LAT]
    qlv = q[:, D_LAT:]
    qs = jnp.exp(0.5 * qlv)
    qm_ref[...] = qm
    qs_ref[...] = qs
    qz_ref[...] = qm + qs * eps_ref[...]

  return pl.pallas_call(
      body,
      out_shape=[
          jax.ShapeDtypeStruct((N, D_LAT), jnp.float32),
          jax.ShapeDtypeStruct((N, D_LAT), jnp.float32),
          jax.ShapeDtypeStruct((N, D_LAT), jnp.float32),
      ],
  )(sp, h2, dinv, Wmv, bmv, eps)


def kernel(x, edge_index, edge_weight, W0, b0, gamma0, beta0,
           W1, b1, gamma1, beta1, Wm, bm, Wv, bv, eps):
  row = edge_index[0].astype(jnp.int32)
  col = edge_index[1].astype(jnp.int32)
  ew = edge_weight.astype(jnp.float32)

  dp = _deg_kernel(col, ew).reshape(NC, N, 1)

  hw0, g0, dinv = _tc1(x, W0, dp)
  s0 = _propagate(g0, row, col, ew, D_HID)

  h1, hw1, g1 = _tc_mid(s0, hw0, dinv, b0, gamma0, beta0, Wnext=W1)
  s1 = _propagate(g1, row, col, ew, D_HID)

  _, h2, gh2 = _tc_mid(s1, hw1, dinv, b1, gamma1, beta1, res=h1)
  s2 = _propagate(gh2, row, col, ew, D_HID)

  Wmv = jnp.concatenate([Wm, Wv], axis=1)
  bmv = jnp.concatenate([bm, bv], axis=0)
  q_z, q_m, q_s = _tc_final(s2, h2, dinv, Wmv, bmv, eps)
  return (q_z, q_m, q_s)
